# Initial kernel scaffold; baseline (speedup 1.0000x reference)
import jax
import jax.numpy as jnp
from jax.experimental import pallas as pl


def kernel(x_block, x_spmt, x_crane, x_facility, edges, batch_block, batch_spmt, batch_crane, batch_facility, params):
    raise NotImplementedError("write your pallas kernel here")


# speedup vs baseline: 1.0008x; 1.0008x over previous
import functools

import jax
import jax.numpy as jnp
from jax.experimental import pallas as pl

_N = {'block': 10000, 'spmt': 512, 'crane': 256, 'facility': 64}
_IN_DIMS = {'block': 8, 'spmt': 10, 'crane': 7, 'facility': 3}
_HID = 256
_HEADS = 4
_CH = _HID // _HEADS
_LAYERS = 2
_EDGE_TYPES = [('block', 'needs_transport', 'spmt', 20000), ('spmt', 'can_transport', 'block', 20000), ('block', 'needs_lift', 'crane', 20000), ('crane', 'can_lift', 'block', 20000), ('block', 'at', 'facility', 10000), ('block', 'precedes', 'block', 100000), ('spmt', 'at', 'facility', 2048), ('crane', 'at', 'facility', 1024)]
_NODE_ORDER = ['block', 'spmt', 'crane', 'facility']


def _ek(s, r, d):
    return s + '__' + r + '__' + d


def _mm_body(x_ref, w_ref, o_ref):
    o_ref[...] = jnp.dot(x_ref[...], w_ref[...], preferred_element_type=jnp.float32)


@functools.partial(jax.jit, static_argnames=('blk',))
def _pallas_mm(x, w, blk=256):
    n, k = x.shape
    m = w.shape[1]
    grid = (pl.cdiv(n, blk),)
    return pl.pallas_call(
        _mm_body,
        grid=grid,
        in_specs=[
            pl.BlockSpec((blk, k), lambda i: (i, 0)),
            pl.BlockSpec((k, m), lambda i: (0, 0)),
        ],
        out_specs=pl.BlockSpec((blk, m), lambda i: (i, 0)),
        out_shape=jax.ShapeDtypeStruct((n, m), jnp.float32),
    )(x, w)


def _leaky(x):
    return jnp.where(x >= 0, x, 0.2 * x)


def _ln(x, g, b):
    m = x.mean(-1, keepdims=True)
    v = ((x - m) ** 2).mean(-1, keepdims=True)
    return (x - m) / jnp.sqrt(v + 1e-5) * g + b


def _seg_softmax(a, d, n):
    amax = jax.ops.segment_max(a, d, num_segments=n)
    amax = jnp.where(jnp.isfinite(amax), amax, 0.0)
    ex = jnp.exp(a - amax[d])
    den = jax.ops.segment_sum(ex, d, num_segments=n)
    return ex / (den[d] + 1e-16)


def kernel(x_block, x_spmt, x_crane, x_facility, edges, batch_block, batch_spmt, batch_crane, batch_facility, params):
    xs = {'block': x_block, 'spmt': x_spmt, 'crane': x_crane, 'facility': x_facility}
    x = {nt: _pallas_mm(xs[nt], params['proj'][nt]['W']) + params['proj'][nt]['b'] for nt in _NODE_ORDER}
    for l in range(_LAYERS):
        acc = {nt: [] for nt in _NODE_ORDER}
        for (s, r, d, ne) in _EDGE_TYPES:
            p = params['layers'][l][_ek(s, r, d)]
            e = edges[_ek(s, r, d)]
            si, di = e[0], e[1]
            hs = _pallas_mm(x[s], p['W']).reshape(_N[s], _HEADS, _CH)
            hd = _pallas_mm(x[d], p['W']).reshape(_N[d], _HEADS, _CH)
            asrc = (hs * p['att_src']).sum(-1)
            adst = (hd * p['att_dst']).sum(-1)
            alpha = _leaky(asrc[si] + adst[di])
            alpha = _seg_softmax(alpha, di, _N[d])
            msg = hs[si] * alpha[:, :, None]
            out = jax.ops.segment_sum(msg, di, num_segments=_N[d]).reshape(_N[d], _HEADS * _CH) + p['bias']
            acc[d].append(out)
        nrm = params['norms'][l]
        newx = {}
        for nt in _NODE_ORDER:
            if acc[nt]:
                xn = jnp.mean(jnp.stack(acc[nt], 0), 0)
            else:
                xn = x[nt]
            newx[nt] = _ln(jax.nn.relu(xn) + x[nt], nrm['g'], nrm['b'])
        x = newx
    pooled = []
    for nt in _NODE_ORDER:
        b = batches = None
        ssum = jnp.sum(x[nt], axis=0, keepdims=True)
        pooled.append(ssum / _N[nt])
    return jnp.concatenate(pooled, axis=-1)


# speedup vs baseline: 13.3845x; 13.3732x over previous
import functools

import jax
import jax.numpy as jnp
from jax import lax
from jax.experimental import pallas as pl
from jax.experimental.pallas import tpu as pltpu
from jax.experimental.pallas import tpu_sc as plsc

_N = {'block': 10000, 'spmt': 512, 'crane': 256, 'facility': 64}
_BLK = {'block': 400, 'spmt': 512, 'crane': 256, 'facility': 64}
_HID = 256
_HEADS = 4
_CH = _HID // _HEADS
_LAYERS = 2
_EDGE_TYPES = [('block', 'needs_transport', 'spmt', 20000), ('spmt', 'can_transport', 'block', 20000), ('block', 'needs_lift', 'crane', 20000), ('crane', 'can_lift', 'block', 20000), ('block', 'at', 'facility', 10000), ('block', 'precedes', 'block', 100000), ('spmt', 'at', 'facility', 2048), ('crane', 'at', 'facility', 1024)]
_NODE_ORDER = ['block', 'spmt', 'crane', 'facility']


def _ek(s, r, d):
    return s + '__' + r + '__' + d


def _rup(x, m):
    return (x + m - 1) // m * m


def _proj_body(x_ref, w_ref, b_ref, o_ref):
    o_ref[...] = jnp.dot(x_ref[...], w_ref[...], preferred_element_type=jnp.float32) + b_ref[...]


def _proj(x, w, b, blk):
    n, k = x.shape
    m = w.shape[1]
    return pl.pallas_call(
        _proj_body,
        grid=(n // blk,),
        in_specs=[
            pl.BlockSpec((blk, k), lambda i: (i, 0)),
            pl.BlockSpec((k, m), lambda i: (0, 0)),
            pl.BlockSpec((1, m), lambda i: (0, 0)),
        ],
        out_specs=pl.BlockSpec((blk, m), lambda i: (i, 0)),
        out_shape=jax.ShapeDtypeStruct((n, m), jnp.float32),
    )(x, w, b)


def _hsc_body(x_ref, w_ref, a_ref, o_ref, o2_ref):
    h = jnp.dot(x_ref[...], w_ref[...], preferred_element_type=jnp.float32)
    o_ref[...] = h
    o2_ref[...] = jnp.dot(h, a_ref[0], preferred_element_type=jnp.float32)


def _hsc(x, w, a3, blk):
    n = x.shape[0]
    nb = n // blk
    return pl.pallas_call(
        _hsc_body,
        grid=(2, nb),
        in_specs=[
            pl.BlockSpec((blk, _HID), lambda c, i: (i, 0)),
            pl.BlockSpec((_HID, 128), lambda c, i: (0, c)),
            pl.BlockSpec((1, 128, 16), lambda c, i: (c, 0, 0)),
        ],
        out_specs=[
            pl.BlockSpec((blk, 128), lambda c, i: (c * nb + i, 0)),
            pl.BlockSpec((blk, 16), lambda c, i: (c * nb + i, 0)),
        ],
        out_shape=[
            jax.ShapeDtypeStruct((2 * n, 128), jnp.float32),
            jax.ShapeDtypeStruct((2 * n, 16), jnp.float32),
        ],
    )(x, w, a3)


def _adst_body(x_ref, w_ref, a_ref, o_ref):
    h = jnp.dot(x_ref[...], w_ref[...], preferred_element_type=jnp.float32)
    o_ref[...] = jnp.dot(h, a_ref[...], preferred_element_type=jnp.float32)


def _adst16(x, w, a16, blk):
    n = x.shape[0]
    return pl.pallas_call(
        _adst_body,
        grid=(n // blk,),
        in_specs=[
            pl.BlockSpec((blk, _HID), lambda i: (i, 0)),
            pl.BlockSpec((_HID, _HID), lambda i: (0, 0)),
            pl.BlockSpec((_HID, 16), lambda i: (0, 0)),
        ],
        out_specs=pl.BlockSpec((blk, 16), lambda i: (i, 0)),
        out_shape=jax.ShapeDtypeStruct((n, 16), jnp.float32),
    )(x, w, a16)


def _combine_body(n_rels, blk, x_ref, g_ref, b_ref, *refs):
    ch_refs = refs[0:2 * n_rels:2]
    den_refs = refs[1:2 * n_rels:2]
    bias_refs = refs[2 * n_rels:3 * n_rels]
    o_ref = refs[3 * n_rels]
    tot = jnp.zeros((blk, _HID), jnp.float32)
    for rr in range(n_rels):
        a = ch_refs[rr][...]
        num = jnp.concatenate([a[0], a[1]], axis=-1)
        d4 = den_refs[rr][...]
        den = jnp.concatenate(
            [jnp.broadcast_to(d4[:, h:h + 1], (blk, _CH)) for h in range(_HEADS)],
            axis=-1)
        tot = tot + num / (den + 1e-16) + bias_refs[rr][...]
    xn = tot * (1.0 / n_rels)
    y = jnp.maximum(xn, 0.0) + x_ref[...]
    m = jnp.mean(y, axis=-1, keepdims=True)
    v = jnp.mean((y - m) ** 2, axis=-1, keepdims=True)
    o_ref[...] = (y - m) * jax.lax.rsqrt(v + 1e-5) * g_ref[...] + b_ref[...]


def _combine(x, g, b, accs, biases, blk):
    n = x.shape[0]
    n_rels = len(accs)
    in_specs = [
        pl.BlockSpec((blk, _HID), lambda i: (i, 0)),
        pl.BlockSpec((1, _HID), lambda i: (0, 0)),
        pl.BlockSpec((1, _HID), lambda i: (0, 0)),
    ]
    flat = []
    for (ch, d4) in accs:
        in_specs.append(pl.BlockSpec((2, blk, 128), lambda i: (0, i, 0)))
        in_specs.append(pl.BlockSpec((blk, 4), lambda i: (i, 0)))
        flat.extend([ch, d4])
    for _ in biases:
        in_specs.append(pl.BlockSpec((1, _HID), lambda i: (0, 0)))
    return pl.pallas_call(
        functools.partial(_combine_body, n_rels, blk),
        grid=(n // blk,),
        in_specs=in_specs,
        out_specs=pl.BlockSpec((blk, _HID), lambda i: (i, 0)),
        out_shape=jax.ShapeDtypeStruct((n, _HID), jnp.float32),
    )(x, g, b, *flat, *biases)


def _pool_body(nb, n, x_ref, o_ref):
    i = pl.program_id(0)

    @pl.when(i == 0)
    def _():
        o_ref[...] = jnp.zeros_like(o_ref)

    o_ref[...] += jnp.sum(x_ref[...], axis=0, keepdims=True)

    @pl.when(i == nb - 1)
    def _():
        o_ref[...] = o_ref[...] * (1.0 / n)


def _pool(x, blk):
    n = x.shape[0]
    nb = n // blk
    return pl.pallas_call(
        functools.partial(_pool_body, nb, n),
        grid=(nb,),
        in_specs=[pl.BlockSpec((blk, _HID), lambda i: (i, 0))],
        out_specs=pl.BlockSpec((1, _HID), lambda i: (0, 0)),
        out_shape=jax.ShapeDtypeStruct((1, _HID), jnp.float32),
    )(x)


@functools.partial(jax.jit, static_argnames=('ne_pad', 'n_s', 'n_d', 'rows_acc'))
def _sc_edge(hsc, asrcf, adstf, sip, dip, z2d, prev, *,
             ne_pad, n_s, n_d, rows_acc):
    e16 = ne_pad // 16
    ng = e16 // 256
    zrows = rows_acc // 16
    nrows = _rup(2 * rows_acc // 128, 16)
    mesh = plsc.VectorSubcoreMesh(core_axis_name="c", subcore_axis_name="s")

    @functools.partial(
        pl.kernel,
        out_type=(
            jax.ShapeDtypeStruct((2, rows_acc, 128), jnp.float32),
            jax.ShapeDtypeStruct((2, nrows, 128), jnp.float32),
        ),
        mesh=mesh,
        compiler_params=pltpu.CompilerParams(needs_layout_passes=False),
        scratch_types=[
            pltpu.VMEM((256,), jnp.int32),
            pltpu.VMEM((256,), jnp.int32),
            pltpu.VMEM((n_s,), jnp.float32),
            pltpu.VMEM((n_s,), jnp.float32),
            pltpu.VMEM((n_d,), jnp.float32),
            pltpu.VMEM((n_d,), jnp.float32),
            pltpu.VMEM((16, 128), jnp.float32),
            pltpu.VMEM((16, 128), jnp.float32),
            pltpu.VMEM((16, 128), jnp.float32),
            pltpu.VMEM_SHARED((rows_acc, 128), jnp.float32),
            pltpu.VMEM_SHARED((nrows, 128), jnp.float32),
            pltpu.SemaphoreType.DMA,
        ],
    )
    def k(hsc_h, asrc_h, adst_h, si_h, di_h, z2_h, prev_h,
          outch_h, outden_h,
          sidx, didx, aev, aod, bev, bod, rows_v, orows_v, ohrows_v,
          acc, dsh, sem1):
        cid = lax.axis_index("c")
        sid = lax.axis_index("s")

        def zero_chunks(dst, total, base):
            off = 0
            while off < total:
                sz = min(64, total - off)
                pltpu.sync_copy(z2_h.at[pl.ds(0, sz)], dst.at[pl.ds(base + off, sz)])
                off += sz

        zero_chunks(acc, zrows, sid * zrows)

        @pl.when(sid == 0)
        def _():
            zero_chunks(dsh, nrows, 0)

        pltpu.sync_copy(asrc_h.at[pl.ds((2 * cid) * n_s, n_s)], aev)
        pltpu.sync_copy(asrc_h.at[pl.ds((2 * cid + 1) * n_s, n_s)], aod)
        pltpu.sync_copy(adst_h.at[pl.ds((2 * cid) * n_d, n_d)], bev)
        pltpu.sync_copy(adst_h.at[pl.ds((2 * cid + 1) * n_d, n_d)], bod)
        plsc.subcore_barrier()

        coff = cid * n_s
        iota = lax.iota(jnp.int32, 16)
        zf = jnp.zeros((16,), jnp.float32)

        def group(g, carry):
            pltpu.sync_copy(si_h.at[pl.ds(sid * e16 + g * 256, 256)], sidx)
            pltpu.sync_copy(di_h.at[pl.ds(sid * e16 + g * 256, 256)], didx)

            def chunk(j, carry2):
                si16 = sidx[pl.ds(j * 16, 16)]
                iv = si16 + coff
                cp = pltpu.async_copy(hsc_h.at[iv], rows_v, sem1)
                sv = didx[pl.ds(j * 16, 16)]
                gv = jnp.minimum(sv, n_d - 1)
                a0 = plsc.load_gather(aev, [si16])
                a1 = plsc.load_gather(aod, [si16])
                b0 = plsc.load_gather(bev, [gv])
                b1 = plsc.load_gather(bod, [gv])
                s0 = a0 + b0
                s0 = jnp.where(s0 >= 0, s0, 0.2 * s0)
                w0 = jnp.exp(s0)
                s1 = a1 + b1
                s1 = jnp.where(s1 >= 0, s1, 0.2 * s1)
                w1 = jnp.exp(s1)
                rv = sv >> 6
                cp.wait()
                for e in range(16):
                    w0s = jnp.broadcast_to(w0[e], (16,))
                    w1s = jnp.broadcast_to(w1[e], (16,))
                    for v in range(8):
                        orows_v[e, pl.ds(v * 16, 16)] = rows_v[e, pl.ds(v * 16, 16)] * (w0s if v < 4 else w1s)
                    sve = sv[e]
                    lane = (sve & 63) * 2
                    vstar = lane >> 4
                    lt = lane & 15
                    pv = jnp.where(iota == lt, w0s, jnp.where(iota == lt + 1, w1s, zf))
                    for v in range(8):
                        ohrows_v[e, pl.ds(v * 16, 16)] = jnp.where(vstar == v, pv, zf)
                pltpu.sync_copy(orows_v, acc.at[sv], add=True)
                pltpu.sync_copy(ohrows_v, dsh.at[rv], add=True)
                return carry2

            lax.fori_loop(0, 16, chunk, 0)
            return carry

        lax.fori_loop(0, ng, group, 0)
        plsc.subcore_barrier()
        pltpu.sync_copy(acc.at[pl.ds(sid * zrows, zrows)],
                        outch_h.at[cid, pl.ds(sid * zrows, zrows)])

        @pl.when(sid == 0)
        def _():
            pltpu.sync_copy(dsh, outden_h.at[cid])

    return k(hsc, asrcf, adstf, sip, dip, z2d, prev)


def _att_src3(att):
    a3 = jnp.zeros((2, 128, 16), jnp.float32)
    for c in range(2):
        for j in range(2):
            a3 = a3.at[c, j * _CH:(j + 1) * _CH, j].set(att[2 * c + j])
    return a3


def _att_dst16(att):
    a = jnp.zeros((_HID, 16), jnp.float32)
    for h in range(_HEADS):
        a = a.at[h * _CH:(h + 1) * _CH, h].set(att[h])
    return a


def kernel(x_block, x_spmt, x_crane, x_facility, edges, batch_block, batch_spmt, batch_crane, batch_facility, params):
    xs = {'block': x_block, 'spmt': x_spmt, 'crane': x_crane, 'facility': x_facility}
    x = {nt: _proj(xs[nt], params['proj'][nt]['W'],
                   params['proj'][nt]['b'].reshape(1, _HID), _BLK[nt])
         for nt in _NODE_ORDER}

    z2d = jnp.zeros((64, 128), jnp.float32)
    prev = z2d[:16]

    eidx = {}
    for (s, r, d, ne) in _EDGE_TYPES:
        key = _ek(s, r, d)
        e = edges[key]
        si, di = e[0], e[1]
        ne_pad = _rup(ne, 4096)
        pad = ne_pad - ne
        si_p = jnp.concatenate([si, jnp.zeros((pad,), jnp.int32)])
        di_p = jnp.concatenate([di, jnp.full((pad,), _N[d], jnp.int32)])
        eidx[key] = (si_p, di_p, ne_pad)

    for l in range(_LAYERS):
        acc = {nt: [] for nt in _NODE_ORDER}
        bias = {nt: [] for nt in _NODE_ORDER}
        for (s, r, d, ne) in _EDGE_TYPES:
            key = _ek(s, r, d)
            p = params['layers'][l][key]
            n_s, n_d = _N[s], _N[d]
            hsc, asrc2 = _hsc(x[s], p['W'], _att_src3(p['att_src']), _BLK[s])
            asrcf = asrc2[:, :2].reshape(2, n_s, 2).transpose(0, 2, 1).reshape(4 * n_s)
            ad16 = _adst16(x[d], p['W'], _att_dst16(p['att_dst']), _BLK[d])
            adstf = ad16[:, :4].T.reshape(4 * n_d)
            si_p, di_p, ne_pad = eidx[key]
            rows_acc = _rup(n_d + 1, 128)
            out_ch, out_den = _sc_edge(hsc, asrcf, adstf, si_p, di_p, z2d, prev,
                                       ne_pad=ne_pad, n_s=n_s, n_d=n_d, rows_acc=rows_acc)
            prev = out_den[0, :16]
            nr = out_den.shape[1]
            den4 = out_den.reshape(2, nr * 128)[:, :2 * rows_acc].reshape(
                2, rows_acc, 2).transpose(1, 0, 2).reshape(rows_acc, 4)
            acc[d].append((out_ch, den4))
            bias[d].append(p['bias'].reshape(1, _HID))
        nrm = params['norms'][l]
        g = nrm['g'].reshape(1, _HID)
        b = nrm['b'].reshape(1, _HID)
        x = {nt: _combine(x[nt], g, b, acc[nt], bias[nt], _BLK[nt])
             for nt in _NODE_ORDER}

    pooled = [_pool(x[nt], _BLK[nt]) for nt in _NODE_ORDER]
    return jnp.concatenate(pooled, axis=-1)


# speedup vs baseline: 17.4121x; 1.3009x over previous
import functools

import jax
import jax.numpy as jnp
from jax import lax
from jax.experimental import pallas as pl
from jax.experimental.pallas import tpu as pltpu
from jax.experimental.pallas import tpu_sc as plsc

_N = {'block': 10000, 'spmt': 512, 'crane': 256, 'facility': 64}
_BLK = {'block': 400, 'spmt': 512, 'crane': 256, 'facility': 64}
_HID = 256
_HEADS = 4
_CH = _HID // _HEADS
_LAYERS = 2
_EDGE_TYPES = [('block', 'needs_transport', 'spmt', 20000), ('spmt', 'can_transport', 'block', 20000), ('block', 'needs_lift', 'crane', 20000), ('crane', 'can_lift', 'block', 20000), ('block', 'at', 'facility', 10000), ('block', 'precedes', 'block', 100000), ('spmt', 'at', 'facility', 2048), ('crane', 'at', 'facility', 1024)]
_NODE_ORDER = ['block', 'spmt', 'crane', 'facility']


def _ek(s, r, d):
    return s + '__' + r + '__' + d


def _rup(x, m):
    return (x + m - 1) // m * m


def _proj_body(x_ref, w_ref, b_ref, o_ref):
    o_ref[...] = jnp.dot(x_ref[...], w_ref[...], preferred_element_type=jnp.float32) + b_ref[...]


def _proj(x, w, b, blk):
    n, k = x.shape
    m = w.shape[1]
    return pl.pallas_call(
        _proj_body,
        grid=(n // blk,),
        in_specs=[
            pl.BlockSpec((blk, k), lambda i: (i, 0)),
            pl.BlockSpec((k, m), lambda i: (0, 0)),
            pl.BlockSpec((1, m), lambda i: (0, 0)),
        ],
        out_specs=pl.BlockSpec((blk, m), lambda i: (i, 0)),
        out_shape=jax.ShapeDtypeStruct((n, m), jnp.float32),
    )(x, w, b)


def _hsc_body(x_ref, w_ref, a_ref, o_ref, o2_ref):
    h = jnp.dot(x_ref[...], w_ref[...], preferred_element_type=jnp.float32)
    o_ref[...] = h
    o2_ref[...] = jnp.dot(h, a_ref[0], preferred_element_type=jnp.float32)


def _hsc(x, w, a3, blk):
    n = x.shape[0]
    nb = n // blk
    return pl.pallas_call(
        _hsc_body,
        grid=(2, nb),
        in_specs=[
            pl.BlockSpec((blk, _HID), lambda c, i: (i, 0)),
            pl.BlockSpec((_HID, 128), lambda c, i: (0, c)),
            pl.BlockSpec((1, 128, 16), lambda c, i: (c, 0, 0)),
        ],
        out_specs=[
            pl.BlockSpec((blk, 128), lambda c, i: (c * nb + i, 0)),
            pl.BlockSpec((blk, 16), lambda c, i: (c * nb + i, 0)),
        ],
        out_shape=[
            jax.ShapeDtypeStruct((2 * n, 128), jnp.float32),
            jax.ShapeDtypeStruct((2 * n, 16), jnp.float32),
        ],
    )(x, w, a3)


def _adst_body(x_ref, w_ref, a_ref, o_ref):
    h = jnp.dot(x_ref[...], w_ref[...], preferred_element_type=jnp.float32)
    o_ref[...] = jnp.dot(h, a_ref[...], preferred_element_type=jnp.float32)


def _adst16(x, w, a16, blk):
    n = x.shape[0]
    return pl.pallas_call(
        _adst_body,
        grid=(n // blk,),
        in_specs=[
            pl.BlockSpec((blk, _HID), lambda i: (i, 0)),
            pl.BlockSpec((_HID, _HID), lambda i: (0, 0)),
            pl.BlockSpec((_HID, 16), lambda i: (0, 0)),
        ],
        out_specs=pl.BlockSpec((blk, 16), lambda i: (i, 0)),
        out_shape=jax.ShapeDtypeStruct((n, 16), jnp.float32),
    )(x, w, a16)


def _combine_body(n_rels, blk, x_ref, g_ref, b_ref, *refs):
    ch_refs = refs[0:2 * n_rels:2]
    den_refs = refs[1:2 * n_rels:2]
    bias_refs = refs[2 * n_rels:3 * n_rels]
    o_ref = refs[3 * n_rels]
    tot = jnp.zeros((blk, _HID), jnp.float32)
    for rr in range(n_rels):
        a = ch_refs[rr][...]
        num = jnp.concatenate([a[0], a[1]], axis=-1)
        d4 = den_refs[rr][...]
        den = jnp.concatenate(
            [jnp.broadcast_to(d4[:, h:h + 1], (blk, _CH)) for h in range(_HEADS)],
            axis=-1)
        tot = tot + num / (den + 1e-16) + bias_refs[rr][...]
    xn = tot * (1.0 / n_rels)
    y = jnp.maximum(xn, 0.0) + x_ref[...]
    m = jnp.mean(y, axis=-1, keepdims=True)
    v = jnp.mean((y - m) ** 2, axis=-1, keepdims=True)
    o_ref[...] = (y - m) * jax.lax.rsqrt(v + 1e-5) * g_ref[...] + b_ref[...]


def _combine(x, g, b, accs, biases, blk):
    n = x.shape[0]
    n_rels = len(accs)
    in_specs = [
        pl.BlockSpec((blk, _HID), lambda i: (i, 0)),
        pl.BlockSpec((1, _HID), lambda i: (0, 0)),
        pl.BlockSpec((1, _HID), lambda i: (0, 0)),
    ]
    flat = []
    for (ch, d4) in accs:
        in_specs.append(pl.BlockSpec((2, blk, 128), lambda i: (0, i, 0)))
        in_specs.append(pl.BlockSpec((blk, 4), lambda i: (i, 0)))
        flat.extend([ch, d4])
    for _ in biases:
        in_specs.append(pl.BlockSpec((1, _HID), lambda i: (0, 0)))
    return pl.pallas_call(
        functools.partial(_combine_body, n_rels, blk),
        grid=(n // blk,),
        in_specs=in_specs,
        out_specs=pl.BlockSpec((blk, _HID), lambda i: (i, 0)),
        out_shape=jax.ShapeDtypeStruct((n, _HID), jnp.float32),
    )(x, g, b, *flat, *biases)


def _pool_body(nb, n, x_ref, o_ref):
    i = pl.program_id(0)

    @pl.when(i == 0)
    def _():
        o_ref[...] = jnp.zeros_like(o_ref)

    o_ref[...] += jnp.sum(x_ref[...], axis=0, keepdims=True)

    @pl.when(i == nb - 1)
    def _():
        o_ref[...] = o_ref[...] * (1.0 / n)


def _pool(x, blk):
    n = x.shape[0]
    nb = n // blk
    return pl.pallas_call(
        functools.partial(_pool_body, nb, n),
        grid=(nb,),
        in_specs=[pl.BlockSpec((blk, _HID), lambda i: (i, 0))],
        out_specs=pl.BlockSpec((1, _HID), lambda i: (0, 0)),
        out_shape=jax.ShapeDtypeStruct((1, _HID), jnp.float32),
    )(x)


@functools.partial(jax.jit, static_argnames=('ne_pad', 'n_s', 'n_d', 'rows_acc'))
def _sc_edge(hsc, asrcf, adstf, sip, dip, z2d, prev, *,
             ne_pad, n_s, n_d, rows_acc):
    e16 = ne_pad // 16
    ng = e16 // 256
    zrows = rows_acc // 16
    nrows = _rup(2 * rows_acc // 128, 16)
    mesh = plsc.VectorSubcoreMesh(core_axis_name="c", subcore_axis_name="s")

    @functools.partial(
        pl.kernel,
        out_type=(
            jax.ShapeDtypeStruct((2, rows_acc, 128), jnp.float32),
            jax.ShapeDtypeStruct((2, nrows, 128), jnp.float32),
        ),
        mesh=mesh,
        compiler_params=pltpu.CompilerParams(needs_layout_passes=False),
        scratch_types=[
            pltpu.VMEM((256,), jnp.int32),
            pltpu.VMEM((256,), jnp.int32),
            pltpu.VMEM((n_s,), jnp.float32),
            pltpu.VMEM((n_s,), jnp.float32),
            pltpu.VMEM((n_d,), jnp.float32),
            pltpu.VMEM((n_d,), jnp.float32),
            pltpu.VMEM((16, 128), jnp.float32),
            pltpu.VMEM((16, 128), jnp.float32),
            pltpu.VMEM((16, 128), jnp.float32),
            pltpu.VMEM_SHARED((rows_acc, 128), jnp.float32),
            pltpu.VMEM_SHARED((nrows, 128), jnp.float32),
            pltpu.SemaphoreType.DMA,
            pltpu.SemaphoreType.DMA,
        ],
    )
    def k(hsc_h, asrc_h, adst_h, si_h, di_h, z2_h, prev_h,
          outch_h, outden_h,
          sidx, didx, aev, aod, bev, bod, rows_v0, rows_v1, orows_v,
          acc, dsh, sem0, sem1):
        cid = lax.axis_index("c")
        sid = lax.axis_index("s")

        def zero_chunks(dst, total, base):
            off = 0
            while off < total:
                sz = min(64, total - off)
                pltpu.sync_copy(z2_h.at[pl.ds(0, sz)], dst.at[pl.ds(base + off, sz)])
                off += sz

        zero_chunks(acc, zrows, sid * zrows)

        @pl.when(sid == 0)
        def _():
            zero_chunks(dsh, nrows, 0)

        pltpu.sync_copy(asrc_h.at[pl.ds((2 * cid) * n_s, n_s)], aev)
        pltpu.sync_copy(asrc_h.at[pl.ds((2 * cid + 1) * n_s, n_s)], aod)
        pltpu.sync_copy(adst_h.at[pl.ds((2 * cid) * n_d, n_d)], bev)
        pltpu.sync_copy(adst_h.at[pl.ds((2 * cid + 1) * n_d, n_d)], bod)
        plsc.subcore_barrier()

        coff = cid * n_s
        iota = lax.iota(jnp.int32, 16)
        zf = jnp.zeros((16,), jnp.float32)

        def issue(j, rbuf, sem):
            si16 = sidx[pl.ds(j * 16, 16)]
            return pltpu.async_copy(hsc_h.at[si16 + coff], rbuf, sem)

        def compute(j, rbuf, sem):
            pltpu.make_async_copy(hsc_h.at[pl.ds(0, 16)], rbuf, sem).wait()
            si16 = sidx[pl.ds(j * 16, 16)]
            sv = didx[pl.ds(j * 16, 16)]
            gv = jnp.minimum(sv, n_d - 1)
            a0 = plsc.load_gather(aev, [si16])
            a1 = plsc.load_gather(aod, [si16])
            b0 = plsc.load_gather(bev, [gv])
            b1 = plsc.load_gather(bod, [gv])
            s0 = a0 + b0
            s0 = jnp.where(s0 >= 0, s0, 0.2 * s0)
            w0 = jnp.exp(s0)
            s1 = a1 + b1
            s1 = jnp.where(s1 >= 0, s1, 0.2 * s1)
            w1 = jnp.exp(s1)
            rv = sv >> 6
            for e in range(16):
                w0s = jnp.broadcast_to(w0[e], (16,))
                w1s = jnp.broadcast_to(w1[e], (16,))
                for v in range(8):
                    orows_v[e, pl.ds(v * 16, 16)] = rbuf[e, pl.ds(v * 16, 16)] * (w0s if v < 4 else w1s)
            pltpu.sync_copy(orows_v, acc.at[sv], add=True)
            for e in range(16):
                w0s = jnp.broadcast_to(w0[e], (16,))
                w1s = jnp.broadcast_to(w1[e], (16,))
                sve = sv[e]
                lane = (sve & 63) * 2
                vstar = lane >> 4
                lt = lane & 15
                pv = jnp.where(iota == lt, w0s, jnp.where(iota == lt + 1, w1s, zf))
                for v in range(8):
                    orows_v[e, pl.ds(v * 16, 16)] = jnp.where(vstar == v, pv, zf)
            pltpu.sync_copy(orows_v, dsh.at[rv], add=True)

        def group(g, carry):
            pltpu.sync_copy(si_h.at[pl.ds(sid * e16 + g * 256, 256)], sidx)
            pltpu.sync_copy(di_h.at[pl.ds(sid * e16 + g * 256, 256)], didx)
            issue(0, rows_v0, sem0)

            def chunk2(j2, carry2):
                issue(2 * j2 + 1, rows_v1, sem1)
                compute(2 * j2, rows_v0, sem0)

                @pl.when(j2 < 7)
                def _():
                    issue(2 * j2 + 2, rows_v0, sem0)

                compute(2 * j2 + 1, rows_v1, sem1)
                return carry2

            lax.fori_loop(0, 8, chunk2, 0)
            return carry

        lax.fori_loop(0, ng, group, 0)
        plsc.subcore_barrier()
        pltpu.sync_copy(acc.at[pl.ds(sid * zrows, zrows)],
                        outch_h.at[cid, pl.ds(sid * zrows, zrows)])

        @pl.when(sid == 0)
        def _():
            pltpu.sync_copy(dsh, outden_h.at[cid])

    return k(hsc, asrcf, adstf, sip, dip, z2d, prev)


def _att_src3(att):
    a3 = jnp.zeros((2, 128, 16), jnp.float32)
    for c in range(2):
        for j in range(2):
            a3 = a3.at[c, j * _CH:(j + 1) * _CH, j].set(att[2 * c + j])
    return a3


def _att_dst16(att):
    a = jnp.zeros((_HID, 16), jnp.float32)
    for h in range(_HEADS):
        a = a.at[h * _CH:(h + 1) * _CH, h].set(att[h])
    return a


def kernel(x_block, x_spmt, x_crane, x_facility, edges, batch_block, batch_spmt, batch_crane, batch_facility, params):
    xs = {'block': x_block, 'spmt': x_spmt, 'crane': x_crane, 'facility': x_facility}
    x = {nt: _proj(xs[nt], params['proj'][nt]['W'],
                   params['proj'][nt]['b'].reshape(1, _HID), _BLK[nt])
         for nt in _NODE_ORDER}

    z2d = jnp.zeros((64, 128), jnp.float32)
    prev = z2d[:16]

    eidx = {}
    for (s, r, d, ne) in _EDGE_TYPES:
        key = _ek(s, r, d)
        e = edges[key]
        si, di = e[0], e[1]
        ne_pad = _rup(ne, 4096)
        pad = ne_pad - ne
        si_p = jnp.concatenate([si, jnp.zeros((pad,), jnp.int32)])
        di_p = jnp.concatenate([di, jnp.full((pad,), _N[d], jnp.int32)])
        eidx[key] = (si_p, di_p, ne_pad)

    for l in range(_LAYERS):
        acc = {nt: [] for nt in _NODE_ORDER}
        bias = {nt: [] for nt in _NODE_ORDER}
        for (s, r, d, ne) in _EDGE_TYPES:
            key = _ek(s, r, d)
            p = params['layers'][l][key]
            n_s, n_d = _N[s], _N[d]
            hsc, asrc2 = _hsc(x[s], p['W'], _att_src3(p['att_src']), _BLK[s])
            asrcf = asrc2[:, :2].reshape(2, n_s, 2).transpose(0, 2, 1).reshape(4 * n_s)
            ad16 = _adst16(x[d], p['W'], _att_dst16(p['att_dst']), _BLK[d])
            adstf = ad16[:, :4].T.reshape(4 * n_d)
            si_p, di_p, ne_pad = eidx[key]
            rows_acc = _rup(n_d + 1, 128)
            out_ch, out_den = _sc_edge(hsc, asrcf, adstf, si_p, di_p, z2d, prev,
                                       ne_pad=ne_pad, n_s=n_s, n_d=n_d, rows_acc=rows_acc)
            prev = out_den[0, :16]
            nr = out_den.shape[1]
            den4 = out_den.reshape(2, nr * 128)[:, :2 * rows_acc].reshape(
                2, rows_acc, 2).transpose(1, 0, 2).reshape(rows_acc, 4)
            acc[d].append((out_ch, den4))
            bias[d].append(p['bias'].reshape(1, _HID))
        nrm = params['norms'][l]
        g = nrm['g'].reshape(1, _HID)
        b = nrm['b'].reshape(1, _HID)
        x = {nt: _combine(x[nt], g, b, acc[nt], bias[nt], _BLK[nt])
             for nt in _NODE_ORDER}

    pooled = [_pool(x[nt], _BLK[nt]) for nt in _NODE_ORDER]
    return jnp.concatenate(pooled, axis=-1)


# speedup vs baseline: 17.5913x; 1.0103x over previous
import functools

import jax
import jax.numpy as jnp
from jax import lax
from jax.experimental import pallas as pl
from jax.experimental.pallas import tpu as pltpu
from jax.experimental.pallas import tpu_sc as plsc

_N = {'block': 10000, 'spmt': 512, 'crane': 256, 'facility': 64}
_BLK = {'block': 400, 'spmt': 512, 'crane': 256, 'facility': 64}
_HID = 256
_HEADS = 4
_CH = _HID // _HEADS
_LAYERS = 2
_EDGE_TYPES = [('block', 'needs_transport', 'spmt', 20000), ('spmt', 'can_transport', 'block', 20000), ('block', 'needs_lift', 'crane', 20000), ('crane', 'can_lift', 'block', 20000), ('block', 'at', 'facility', 10000), ('block', 'precedes', 'block', 100000), ('spmt', 'at', 'facility', 2048), ('crane', 'at', 'facility', 1024)]
_NODE_ORDER = ['block', 'spmt', 'crane', 'facility']


def _ek(s, r, d):
    return s + '__' + r + '__' + d


def _rup(x, m):
    return (x + m - 1) // m * m


def _proj_body(x_ref, w_ref, b_ref, o_ref):
    o_ref[...] = jnp.dot(x_ref[...], w_ref[...], preferred_element_type=jnp.float32) + b_ref[...]


def _proj(x, w, b, blk):
    n, k = x.shape
    m = w.shape[1]
    return pl.pallas_call(
        _proj_body,
        grid=(n // blk,),
        in_specs=[
            pl.BlockSpec((blk, k), lambda i: (i, 0)),
            pl.BlockSpec((k, m), lambda i: (0, 0)),
            pl.BlockSpec((1, m), lambda i: (0, 0)),
        ],
        out_specs=pl.BlockSpec((blk, m), lambda i: (i, 0)),
        out_shape=jax.ShapeDtypeStruct((n, m), jnp.float32),
    )(x, w, b)


def _hsc_body(x_ref, w_ref, a_ref, o_ref, o2_ref):
    h = jnp.dot(x_ref[...], w_ref[...], preferred_element_type=jnp.float32)
    o_ref[...] = h
    o2_ref[...] = jnp.dot(h, a_ref[0], preferred_element_type=jnp.float32)


def _hsc(x, w, a3, blk):
    n = x.shape[0]
    nb = n // blk
    return pl.pallas_call(
        _hsc_body,
        grid=(2, nb),
        in_specs=[
            pl.BlockSpec((blk, _HID), lambda c, i: (i, 0)),
            pl.BlockSpec((_HID, 128), lambda c, i: (0, c)),
            pl.BlockSpec((1, 128, 16), lambda c, i: (c, 0, 0)),
        ],
        out_specs=[
            pl.BlockSpec((blk, 128), lambda c, i: (c * nb + i, 0)),
            pl.BlockSpec((blk, 16), lambda c, i: (c * nb + i, 0)),
        ],
        out_shape=[
            jax.ShapeDtypeStruct((2 * n, 128), jnp.float32),
            jax.ShapeDtypeStruct((2 * n, 16), jnp.float32),
        ],
    )(x, w, a3)


def _adst_body(x_ref, w_ref, a_ref, o_ref):
    h = jnp.dot(x_ref[...], w_ref[...], preferred_element_type=jnp.float32)
    o_ref[...] = jnp.dot(h, a_ref[...], preferred_element_type=jnp.float32)


def _adst16(x, w, a16, blk):
    n = x.shape[0]
    return pl.pallas_call(
        _adst_body,
        grid=(n // blk,),
        in_specs=[
            pl.BlockSpec((blk, _HID), lambda i: (i, 0)),
            pl.BlockSpec((_HID, _HID), lambda i: (0, 0)),
            pl.BlockSpec((_HID, 16), lambda i: (0, 0)),
        ],
        out_specs=pl.BlockSpec((blk, 16), lambda i: (i, 0)),
        out_shape=jax.ShapeDtypeStruct((n, 16), jnp.float32),
    )(x, w, a16)


def _combine_body(n_rels, blk, x_ref, g_ref, b_ref, *refs):
    ch_refs = refs[0:2 * n_rels:2]
    den_refs = refs[1:2 * n_rels:2]
    bias_refs = refs[2 * n_rels:3 * n_rels]
    o_ref = refs[3 * n_rels]
    tot = jnp.zeros((blk, _HID), jnp.float32)
    for rr in range(n_rels):
        a = ch_refs[rr][...]
        num = jnp.concatenate([a[0], a[1]], axis=-1)
        d4 = den_refs[rr][...]
        den = jnp.concatenate(
            [jnp.broadcast_to(d4[:, h:h + 1], (blk, _CH)) for h in range(_HEADS)],
            axis=-1)
        tot = tot + num / (den + 1e-16) + bias_refs[rr][...]
    xn = tot * (1.0 / n_rels)
    y = jnp.maximum(xn, 0.0) + x_ref[...]
    m = jnp.mean(y, axis=-1, keepdims=True)
    v = jnp.mean((y - m) ** 2, axis=-1, keepdims=True)
    o_ref[...] = (y - m) * jax.lax.rsqrt(v + 1e-5) * g_ref[...] + b_ref[...]


def _combine(x, g, b, accs, biases, blk):
    n = x.shape[0]
    n_rels = len(accs)
    in_specs = [
        pl.BlockSpec((blk, _HID), lambda i: (i, 0)),
        pl.BlockSpec((1, _HID), lambda i: (0, 0)),
        pl.BlockSpec((1, _HID), lambda i: (0, 0)),
    ]
    flat = []
    for (ch, d4) in accs:
        in_specs.append(pl.BlockSpec((2, blk, 128), lambda i: (0, i, 0)))
        in_specs.append(pl.BlockSpec((blk, 4), lambda i: (i, 0)))
        flat.extend([ch, d4])
    for _ in biases:
        in_specs.append(pl.BlockSpec((1, _HID), lambda i: (0, 0)))
    return pl.pallas_call(
        functools.partial(_combine_body, n_rels, blk),
        grid=(n // blk,),
        in_specs=in_specs,
        out_specs=pl.BlockSpec((blk, _HID), lambda i: (i, 0)),
        out_shape=jax.ShapeDtypeStruct((n, _HID), jnp.float32),
    )(x, g, b, *flat, *biases)


def _pool_body(nb, n, x_ref, o_ref):
    i = pl.program_id(0)

    @pl.when(i == 0)
    def _():
        o_ref[...] = jnp.zeros_like(o_ref)

    o_ref[...] += jnp.sum(x_ref[...], axis=0, keepdims=True)

    @pl.when(i == nb - 1)
    def _():
        o_ref[...] = o_ref[...] * (1.0 / n)


def _pool(x, blk):
    n = x.shape[0]
    nb = n // blk
    return pl.pallas_call(
        functools.partial(_pool_body, nb, n),
        grid=(nb,),
        in_specs=[pl.BlockSpec((blk, _HID), lambda i: (i, 0))],
        out_specs=pl.BlockSpec((1, _HID), lambda i: (0, 0)),
        out_shape=jax.ShapeDtypeStruct((1, _HID), jnp.float32),
    )(x)


@functools.partial(jax.jit, static_argnames=('ne_pad', 'n_s', 'n_d', 'rows_acc'))
def _sc_edge(hsc, asrcf, adstf, sip, dip, z2d, prev, *,
             ne_pad, n_s, n_d, rows_acc):
    e16 = ne_pad // 16
    ng = e16 // 256
    zrows = rows_acc // 16
    nrows = _rup(2 * rows_acc // 128, 16)
    mesh = plsc.VectorSubcoreMesh(core_axis_name="c", subcore_axis_name="s")

    @functools.partial(
        pl.kernel,
        out_type=(
            jax.ShapeDtypeStruct((2, rows_acc, 128), jnp.float32),
            jax.ShapeDtypeStruct((2, nrows, 128), jnp.float32),
        ),
        mesh=mesh,
        compiler_params=pltpu.CompilerParams(needs_layout_passes=False),
        scratch_types=[
            pltpu.VMEM((256,), jnp.int32),
            pltpu.VMEM((256,), jnp.int32),
            pltpu.VMEM((n_s,), jnp.float32),
            pltpu.VMEM((n_s,), jnp.float32),
            pltpu.VMEM((n_d,), jnp.float32),
            pltpu.VMEM((n_d,), jnp.float32),
            pltpu.VMEM((16, 128), jnp.float32),
            pltpu.VMEM((16, 128), jnp.float32),
            pltpu.VMEM((16, 128), jnp.float32),
            pltpu.VMEM_SHARED((rows_acc, 128), jnp.float32),
            pltpu.VMEM_SHARED((nrows, 128), jnp.float32),
            pltpu.SemaphoreType.DMA,
            pltpu.SemaphoreType.DMA,
        ],
    )
    def k(hsc_h, asrc_h, adst_h, si_h, di_h, z2_h, prev_h,
          outch_h, outden_h,
          sidx, didx, aev, aod, bev, bod, rows_v0, rows_v1, orows_v,
          acc, dsh, sem0, sem1):
        cid = lax.axis_index("c")
        sid = lax.axis_index("s")

        pend = []

        def zero_chunks(dst, total, base):
            off = 0
            while off < total:
                sz = min(64, total - off)
                pend.append(pltpu.async_copy(
                    z2_h.at[pl.ds(0, sz)], dst.at[pl.ds(base + off, sz)], sem0))
                off += sz

        zero_chunks(acc, zrows, sid * zrows)

        @pl.when(sid == 0)
        def _():
            off = 0
            while off < nrows:
                sz = min(64, nrows - off)
                pltpu.async_copy(z2_h.at[pl.ds(0, sz)],
                                 dsh.at[pl.ds(off, sz)], sem1).wait()
                off += sz

        pend.append(pltpu.async_copy(asrc_h.at[pl.ds((2 * cid) * n_s, n_s)], aev, sem0))
        pend.append(pltpu.async_copy(asrc_h.at[pl.ds((2 * cid + 1) * n_s, n_s)], aod, sem0))
        pend.append(pltpu.async_copy(adst_h.at[pl.ds((2 * cid) * n_d, n_d)], bev, sem0))
        pend.append(pltpu.async_copy(adst_h.at[pl.ds((2 * cid + 1) * n_d, n_d)], bod, sem0))
        for cp in pend:
            cp.wait()
        plsc.subcore_barrier()

        coff = cid * n_s
        iota = lax.iota(jnp.int32, 16)
        zf = jnp.zeros((16,), jnp.float32)

        def issue(j, rbuf, sem):
            si16 = sidx[pl.ds(j * 16, 16)]
            return pltpu.async_copy(hsc_h.at[si16 + coff], rbuf, sem)

        def compute(j, rbuf, sem):
            pltpu.make_async_copy(hsc_h.at[pl.ds(0, 16)], rbuf, sem).wait()
            si16 = sidx[pl.ds(j * 16, 16)]
            sv = didx[pl.ds(j * 16, 16)]
            gv = jnp.minimum(sv, n_d - 1)
            a0 = plsc.load_gather(aev, [si16])
            a1 = plsc.load_gather(aod, [si16])
            b0 = plsc.load_gather(bev, [gv])
            b1 = plsc.load_gather(bod, [gv])
            s0 = a0 + b0
            s0 = jnp.where(s0 >= 0, s0, 0.2 * s0)
            w0 = jnp.exp(s0)
            s1 = a1 + b1
            s1 = jnp.where(s1 >= 0, s1, 0.2 * s1)
            w1 = jnp.exp(s1)
            rv = sv >> 6
            for e in range(16):
                w0s = jnp.broadcast_to(w0[e], (16,))
                w1s = jnp.broadcast_to(w1[e], (16,))
                for v in range(8):
                    orows_v[e, pl.ds(v * 16, 16)] = rbuf[e, pl.ds(v * 16, 16)] * (w0s if v < 4 else w1s)
            pltpu.sync_copy(orows_v, acc.at[sv], add=True)
            for e in range(16):
                w0s = jnp.broadcast_to(w0[e], (16,))
                w1s = jnp.broadcast_to(w1[e], (16,))
                sve = sv[e]
                lane = (sve & 63) * 2
                vstar = lane >> 4
                lt = lane & 15
                pv = jnp.where(iota == lt, w0s, jnp.where(iota == lt + 1, w1s, zf))
                for v in range(8):
                    orows_v[e, pl.ds(v * 16, 16)] = jnp.where(vstar == v, pv, zf)
            pltpu.sync_copy(orows_v, dsh.at[rv], add=True)

        def group(g, carry):
            pltpu.sync_copy(si_h.at[pl.ds(sid * e16 + g * 256, 256)], sidx)
            pltpu.sync_copy(di_h.at[pl.ds(sid * e16 + g * 256, 256)], didx)
            issue(0, rows_v0, sem0)

            def chunk2(j2, carry2):
                issue(2 * j2 + 1, rows_v1, sem1)
                compute(2 * j2, rows_v0, sem0)

                @pl.when(j2 < 7)
                def _():
                    issue(2 * j2 + 2, rows_v0, sem0)

                compute(2 * j2 + 1, rows_v1, sem1)
                return carry2

            lax.fori_loop(0, 8, chunk2, 0)
            return carry

        lax.fori_loop(0, ng, group, 0)
        plsc.subcore_barrier()
        pltpu.sync_copy(acc.at[pl.ds(sid * zrows, zrows)],
                        outch_h.at[cid, pl.ds(sid * zrows, zrows)])

        @pl.when(sid == 0)
        def _():
            pltpu.sync_copy(dsh, outden_h.at[cid])

    return k(hsc, asrcf, adstf, sip, dip, z2d, prev)


def _att_src3(att):
    a3 = jnp.zeros((2, 128, 16), jnp.float32)
    for c in range(2):
        for j in range(2):
            a3 = a3.at[c, j * _CH:(j + 1) * _CH, j].set(att[2 * c + j])
    return a3


def _att_dst16(att):
    a = jnp.zeros((_HID, 16), jnp.float32)
    for h in range(_HEADS):
        a = a.at[h * _CH:(h + 1) * _CH, h].set(att[h])
    return a


def kernel(x_block, x_spmt, x_crane, x_facility, edges, batch_block, batch_spmt, batch_crane, batch_facility, params):
    xs = {'block': x_block, 'spmt': x_spmt, 'crane': x_crane, 'facility': x_facility}
    x = {nt: _proj(xs[nt], params['proj'][nt]['W'],
                   params['proj'][nt]['b'].reshape(1, _HID), _BLK[nt])
         for nt in _NODE_ORDER}

    z2d = jnp.zeros((64, 128), jnp.float32)
    prev = z2d[:16]

    eidx = {}
    for (s, r, d, ne) in _EDGE_TYPES:
        key = _ek(s, r, d)
        e = edges[key]
        si, di = e[0], e[1]
        ne_pad = _rup(ne, 4096)
        pad = ne_pad - ne
        si_p = jnp.concatenate([si, jnp.zeros((pad,), jnp.int32)])
        di_p = jnp.concatenate([di, jnp.full((pad,), _N[d], jnp.int32)])
        eidx[key] = (si_p, di_p, ne_pad)

    for l in range(_LAYERS):
        acc = {nt: [] for nt in _NODE_ORDER}
        bias = {nt: [] for nt in _NODE_ORDER}
        for (s, r, d, ne) in _EDGE_TYPES:
            key = _ek(s, r, d)
            p = params['layers'][l][key]
            n_s, n_d = _N[s], _N[d]
            hsc, asrc2 = _hsc(x[s], p['W'], _att_src3(p['att_src']), _BLK[s])
            asrcf = asrc2[:, :2].reshape(2, n_s, 2).transpose(0, 2, 1).reshape(4 * n_s)
            ad16 = _adst16(x[d], p['W'], _att_dst16(p['att_dst']), _BLK[d])
            adstf = ad16[:, :4].T.reshape(4 * n_d)
            si_p, di_p, ne_pad = eidx[key]
            rows_acc = _rup(n_d + 1, 128)
            out_ch, out_den = _sc_edge(hsc, asrcf, adstf, si_p, di_p, z2d, prev,
                                       ne_pad=ne_pad, n_s=n_s, n_d=n_d, rows_acc=rows_acc)
            prev = out_den[0, :16]
            nr = out_den.shape[1]
            den4 = out_den.reshape(2, nr * 128)[:, :2 * rows_acc].reshape(
                2, rows_acc, 2).transpose(1, 0, 2).reshape(rows_acc, 4)
            acc[d].append((out_ch, den4))
            bias[d].append(p['bias'].reshape(1, _HID))
        nrm = params['norms'][l]
        g = nrm['g'].reshape(1, _HID)
        b = nrm['b'].reshape(1, _HID)
        x = {nt: _combine(x[nt], g, b, acc[nt], bias[nt], _BLK[nt])
             for nt in _NODE_ORDER}

    pooled = [_pool(x[nt], _BLK[nt]) for nt in _NODE_ORDER]
    return jnp.concatenate(pooled, axis=-1)


# speedup vs baseline: 17.8241x; 1.0132x over previous
import functools

import jax
import jax.numpy as jnp
from jax import lax
from jax.experimental import pallas as pl
from jax.experimental.pallas import tpu as pltpu
from jax.experimental.pallas import tpu_sc as plsc

_N = {'block': 10000, 'spmt': 512, 'crane': 256, 'facility': 64}
_BLK = {'block': 400, 'spmt': 512, 'crane': 256, 'facility': 64}
_HID = 256
_HEADS = 4
_CH = _HID // _HEADS
_LAYERS = 2
_EDGE_TYPES = [('block', 'needs_transport', 'spmt', 20000), ('spmt', 'can_transport', 'block', 20000), ('block', 'needs_lift', 'crane', 20000), ('crane', 'can_lift', 'block', 20000), ('block', 'at', 'facility', 10000), ('block', 'precedes', 'block', 100000), ('spmt', 'at', 'facility', 2048), ('crane', 'at', 'facility', 1024)]
_NODE_ORDER = ['block', 'spmt', 'crane', 'facility']


def _ek(s, r, d):
    return s + '__' + r + '__' + d


def _rup(x, m):
    return (x + m - 1) // m * m


def _proj_body(x_ref, w_ref, b_ref, o_ref):
    o_ref[...] = jnp.dot(x_ref[...], w_ref[...], preferred_element_type=jnp.float32) + b_ref[...]


def _proj(x, w, b, blk):
    n, k = x.shape
    m = w.shape[1]
    return pl.pallas_call(
        _proj_body,
        grid=(n // blk,),
        in_specs=[
            pl.BlockSpec((blk, k), lambda i: (i, 0)),
            pl.BlockSpec((k, m), lambda i: (0, 0)),
            pl.BlockSpec((1, m), lambda i: (0, 0)),
        ],
        out_specs=pl.BlockSpec((blk, m), lambda i: (i, 0)),
        out_shape=jax.ShapeDtypeStruct((n, m), jnp.float32),
    )(x, w, b)


def _hsc_body(x_ref, w_ref, a_ref, o_ref, o2_ref):
    h = jnp.dot(x_ref[...], w_ref[...], preferred_element_type=jnp.float32)
    o_ref[...] = h
    o2_ref[...] = jnp.dot(h, a_ref[0], preferred_element_type=jnp.float32)


def _hsc(x, w, a3, blk):
    n = x.shape[0]
    nb = n // blk
    return pl.pallas_call(
        _hsc_body,
        grid=(2, nb),
        in_specs=[
            pl.BlockSpec((blk, _HID), lambda c, i: (i, 0)),
            pl.BlockSpec((_HID, 128), lambda c, i: (0, c)),
            pl.BlockSpec((1, 128, 16), lambda c, i: (c, 0, 0)),
        ],
        out_specs=[
            pl.BlockSpec((blk, 128), lambda c, i: (c * nb + i, 0)),
            pl.BlockSpec((blk, 16), lambda c, i: (c * nb + i, 0)),
        ],
        out_shape=[
            jax.ShapeDtypeStruct((2 * n, 128), jnp.float32),
            jax.ShapeDtypeStruct((2 * n, 16), jnp.float32),
        ],
    )(x, w, a3)


def _adst_body(x_ref, w_ref, a_ref, o_ref):
    h = jnp.dot(x_ref[...], w_ref[...], preferred_element_type=jnp.float32)
    o_ref[...] = jnp.dot(h, a_ref[...], preferred_element_type=jnp.float32)


def _adst16(x, w, a16, blk):
    n = x.shape[0]
    return pl.pallas_call(
        _adst_body,
        grid=(n // blk,),
        in_specs=[
            pl.BlockSpec((blk, _HID), lambda i: (i, 0)),
            pl.BlockSpec((_HID, _HID), lambda i: (0, 0)),
            pl.BlockSpec((_HID, 16), lambda i: (0, 0)),
        ],
        out_specs=pl.BlockSpec((blk, 16), lambda i: (i, 0)),
        out_shape=jax.ShapeDtypeStruct((n, 16), jnp.float32),
    )(x, w, a16)


def _combine_body(n_rels, blk, x_ref, g_ref, b_ref, *refs):
    ch_refs = refs[0:2 * n_rels:2]
    den_refs = refs[1:2 * n_rels:2]
    bias_refs = refs[2 * n_rels:3 * n_rels]
    o_ref = refs[3 * n_rels]
    tot = jnp.zeros((blk, _HID), jnp.float32)
    for rr in range(n_rels):
        a = ch_refs[rr][...]
        num = jnp.concatenate([a[0], a[1]], axis=-1)
        d4 = den_refs[rr][...]
        den = jnp.concatenate(
            [jnp.broadcast_to(d4[:, h:h + 1], (blk, _CH)) for h in range(_HEADS)],
            axis=-1)
        tot = tot + num / (den + 1e-16) + bias_refs[rr][...]
    xn = tot * (1.0 / n_rels)
    y = jnp.maximum(xn, 0.0) + x_ref[...]
    m = jnp.mean(y, axis=-1, keepdims=True)
    v = jnp.mean((y - m) ** 2, axis=-1, keepdims=True)
    o_ref[...] = (y - m) * jax.lax.rsqrt(v + 1e-5) * g_ref[...] + b_ref[...]


def _combine(x, g, b, accs, biases, blk):
    n = x.shape[0]
    n_rels = len(accs)
    in_specs = [
        pl.BlockSpec((blk, _HID), lambda i: (i, 0)),
        pl.BlockSpec((1, _HID), lambda i: (0, 0)),
        pl.BlockSpec((1, _HID), lambda i: (0, 0)),
    ]
    flat = []
    for (ch, d4) in accs:
        in_specs.append(pl.BlockSpec((2, blk, 128), lambda i: (0, i, 0)))
        in_specs.append(pl.BlockSpec((blk, 4), lambda i: (i, 0)))
        flat.extend([ch, d4])
    for _ in biases:
        in_specs.append(pl.BlockSpec((1, _HID), lambda i: (0, 0)))
    return pl.pallas_call(
        functools.partial(_combine_body, n_rels, blk),
        grid=(n // blk,),
        in_specs=in_specs,
        out_specs=pl.BlockSpec((blk, _HID), lambda i: (i, 0)),
        out_shape=jax.ShapeDtypeStruct((n, _HID), jnp.float32),
    )(x, g, b, *flat, *biases)


def _pool_body(nb, n, x_ref, o_ref):
    i = pl.program_id(0)

    @pl.when(i == 0)
    def _():
        o_ref[...] = jnp.zeros_like(o_ref)

    o_ref[...] += jnp.sum(x_ref[...], axis=0, keepdims=True)

    @pl.when(i == nb - 1)
    def _():
        o_ref[...] = o_ref[...] * (1.0 / n)


def _pool(x, blk):
    n = x.shape[0]
    nb = n // blk
    return pl.pallas_call(
        functools.partial(_pool_body, nb, n),
        grid=(nb,),
        in_specs=[pl.BlockSpec((blk, _HID), lambda i: (i, 0))],
        out_specs=pl.BlockSpec((1, _HID), lambda i: (0, 0)),
        out_shape=jax.ShapeDtypeStruct((1, _HID), jnp.float32),
    )(x)


@functools.partial(jax.jit, static_argnames=('ne_pad', 'n_s', 'n_d', 'rows_acc', 'dbuf'))
def _sc_edge(hsc, asrcf, adstf, sip, dip, z2d, prev, *,
             ne_pad, n_s, n_d, rows_acc, dbuf):
    e16 = ne_pad // 16
    ng = e16 // 256
    zrows = rows_acc // 16
    nrows = _rup(2 * rows_acc // 128, 16)
    mesh = plsc.VectorSubcoreMesh(core_axis_name="c", subcore_axis_name="s")

    @functools.partial(
        pl.kernel,
        out_type=(
            jax.ShapeDtypeStruct((2, rows_acc, 128), jnp.float32),
            jax.ShapeDtypeStruct((2, nrows, 128), jnp.float32),
        ),
        mesh=mesh,
        compiler_params=pltpu.CompilerParams(needs_layout_passes=False),
        scratch_types=[
            pltpu.VMEM((256,), jnp.int32),
            pltpu.VMEM((256,), jnp.int32),
            pltpu.VMEM((n_s,), jnp.float32),
            pltpu.VMEM((n_s,), jnp.float32),
            pltpu.VMEM((n_d,), jnp.float32),
            pltpu.VMEM((n_d,), jnp.float32),
            pltpu.VMEM((16, 128), jnp.float32),
            pltpu.VMEM((16, 128), jnp.float32),
            pltpu.VMEM((16, 128), jnp.float32),
            pltpu.VMEM((16, 128) if dbuf else (16,), jnp.float32),
            pltpu.VMEM((16, 128) if dbuf else (16,), jnp.float32),
            pltpu.VMEM_SHARED((rows_acc, 128), jnp.float32),
            pltpu.VMEM_SHARED((nrows, 128), jnp.float32),
            pltpu.SemaphoreType.DMA,
            pltpu.SemaphoreType.DMA,
            pltpu.SemaphoreType.DMA,
            pltpu.SemaphoreType.DMA,
        ],
    )
    def k(hsc_h, asrc_h, adst_h, si_h, di_h, z2_h, prev_h,
          outch_h, outden_h,
          sidx, didx, aev, aod, bev, bod, rows_v0, rows_v1, orows_v, orows_b,
          ohrows_v, acc, dsh, sem0, sem1, semA, semB):
        cid = lax.axis_index("c")
        sid = lax.axis_index("s")

        pend = []

        def zero_chunks(dst, total, base):
            off = 0
            while off < total:
                sz = min(64, total - off)
                pend.append(pltpu.async_copy(
                    z2_h.at[pl.ds(0, sz)], dst.at[pl.ds(base + off, sz)], sem0))
                off += sz

        zero_chunks(acc, zrows, sid * zrows)

        @pl.when(sid == 0)
        def _():
            off = 0
            while off < nrows:
                sz = min(64, nrows - off)
                pltpu.async_copy(z2_h.at[pl.ds(0, sz)],
                                 dsh.at[pl.ds(off, sz)], sem1).wait()
                off += sz

        pend.append(pltpu.async_copy(asrc_h.at[pl.ds((2 * cid) * n_s, n_s)], aev, sem0))
        pend.append(pltpu.async_copy(asrc_h.at[pl.ds((2 * cid + 1) * n_s, n_s)], aod, sem0))
        pend.append(pltpu.async_copy(adst_h.at[pl.ds((2 * cid) * n_d, n_d)], bev, sem0))
        pend.append(pltpu.async_copy(adst_h.at[pl.ds((2 * cid + 1) * n_d, n_d)], bod, sem0))
        for cp in pend:
            cp.wait()
        plsc.subcore_barrier()

        coff = cid * n_s
        iota = lax.iota(jnp.int32, 16)
        zf = jnp.zeros((16,), jnp.float32)

        def issue(j, rbuf, sem):
            si16 = sidx[pl.ds(j * 16, 16)]
            return pltpu.async_copy(hsc_h.at[si16 + coff], rbuf, sem)

        def compute(j, rbuf, sem, obuf, osem):
            pltpu.make_async_copy(hsc_h.at[pl.ds(0, 16)], rbuf, sem).wait()
            si16 = sidx[pl.ds(j * 16, 16)]
            sv = didx[pl.ds(j * 16, 16)]
            gv = jnp.minimum(sv, n_d - 1)
            a0 = plsc.load_gather(aev, [si16])
            a1 = plsc.load_gather(aod, [si16])
            b0 = plsc.load_gather(bev, [gv])
            b1 = plsc.load_gather(bod, [gv])
            s0 = a0 + b0
            s0 = jnp.where(s0 >= 0, s0, 0.2 * s0)
            w0 = jnp.exp(s0)
            s1 = a1 + b1
            s1 = jnp.where(s1 >= 0, s1, 0.2 * s1)
            w1 = jnp.exp(s1)
            rv = sv >> 6
            for e in range(16):
                w0s = jnp.broadcast_to(w0[e], (16,))
                w1s = jnp.broadcast_to(w1[e], (16,))
                for v in range(8):
                    obuf[e, pl.ds(v * 16, 16)] = rbuf[e, pl.ds(v * 16, 16)] * (w0s if v < 4 else w1s)
            if dbuf:
                pltpu.async_copy(obuf, acc.at[sv], semA if osem == 0 else semB, add=True)
                dhb = ohrows_v
            else:
                pltpu.sync_copy(obuf, acc.at[sv], add=True)
                dhb = obuf
            for e in range(16):
                w0s = jnp.broadcast_to(w0[e], (16,))
                w1s = jnp.broadcast_to(w1[e], (16,))
                sve = sv[e]
                lane = (sve & 63) * 2
                vstar = lane >> 4
                lt = lane & 15
                pv = jnp.where(iota == lt, w0s, jnp.where(iota == lt + 1, w1s, zf))
                for v in range(8):
                    dhb[e, pl.ds(v * 16, 16)] = jnp.where(vstar == v, pv, zf)
            pltpu.sync_copy(dhb, dsh.at[rv], add=True)

        def drain(sem):
            pltpu.make_async_copy(hsc_h.at[pl.ds(0, 16)], orows_v, sem).wait()

        def group(g, carry):
            pltpu.sync_copy(si_h.at[pl.ds(sid * e16 + g * 256, 256)], sidx)
            pltpu.sync_copy(di_h.at[pl.ds(sid * e16 + g * 256, 256)], didx)
            issue(0, rows_v0, sem0)

            def chunk2(j2, carry2):
                issue(2 * j2 + 1, rows_v1, sem1)
                if dbuf:
                    @pl.when(j2 > 0)
                    def _():
                        drain(semA)
                compute(2 * j2, rows_v0, sem0, orows_v, 0)

                @pl.when(j2 < 7)
                def _():
                    issue(2 * j2 + 2, rows_v0, sem0)

                if dbuf:
                    @pl.when(j2 > 0)
                    def _():
                        drain(semB)
                compute(2 * j2 + 1, rows_v1, sem1, orows_b if dbuf else orows_v, 1)
                return carry2

            lax.fori_loop(0, 8, chunk2, 0)
            if dbuf:
                drain(semA)
                drain(semB)
            return carry

        lax.fori_loop(0, ng, group, 0)
        plsc.subcore_barrier()
        pltpu.sync_copy(acc.at[pl.ds(sid * zrows, zrows)],
                        outch_h.at[cid, pl.ds(sid * zrows, zrows)])

        @pl.when(sid == 0)
        def _():
            pltpu.sync_copy(dsh, outden_h.at[cid])

    return k(hsc, asrcf, adstf, sip, dip, z2d, prev)


def _att_src3(att):
    a3 = jnp.zeros((2, 128, 16), jnp.float32)
    for c in range(2):
        for j in range(2):
            a3 = a3.at[c, j * _CH:(j + 1) * _CH, j].set(att[2 * c + j])
    return a3


def _att_dst16(att):
    a = jnp.zeros((_HID, 16), jnp.float32)
    for h in range(_HEADS):
        a = a.at[h * _CH:(h + 1) * _CH, h].set(att[h])
    return a


def kernel(x_block, x_spmt, x_crane, x_facility, edges, batch_block, batch_spmt, batch_crane, batch_facility, params):
    xs = {'block': x_block, 'spmt': x_spmt, 'crane': x_crane, 'facility': x_facility}
    x = {nt: _proj(xs[nt], params['proj'][nt]['W'],
                   params['proj'][nt]['b'].reshape(1, _HID), _BLK[nt])
         for nt in _NODE_ORDER}

    z2d = jnp.zeros((64, 128), jnp.float32)
    prev = z2d[:16]

    eidx = {}
    for (s, r, d, ne) in _EDGE_TYPES:
        key = _ek(s, r, d)
        e = edges[key]
        si, di = e[0], e[1]
        ne_pad = _rup(ne, 4096)
        pad = ne_pad - ne
        si_p = jnp.concatenate([si, jnp.zeros((pad,), jnp.int32)])
        di_p = jnp.concatenate([di, jnp.full((pad,), _N[d], jnp.int32)])
        eidx[key] = (si_p, di_p, ne_pad)

    for l in range(_LAYERS):
        acc = {nt: [] for nt in _NODE_ORDER}
        bias = {nt: [] for nt in _NODE_ORDER}
        for (s, r, d, ne) in _EDGE_TYPES:
            key = _ek(s, r, d)
            p = params['layers'][l][key]
            n_s, n_d = _N[s], _N[d]
            hsc, asrc2 = _hsc(x[s], p['W'], _att_src3(p['att_src']), _BLK[s])
            asrcf = asrc2[:, :2].reshape(2, n_s, 2).transpose(0, 2, 1).reshape(4 * n_s)
            ad16 = _adst16(x[d], p['W'], _att_dst16(p['att_dst']), _BLK[d])
            adstf = ad16[:, :4].T.reshape(4 * n_d)
            si_p, di_p, ne_pad = eidx[key]
            rows_acc = _rup(n_d + 1, 128)
            out_ch, out_den = _sc_edge(hsc, asrcf, adstf, si_p, di_p, z2d, prev,
                                       ne_pad=ne_pad, n_s=n_s, n_d=n_d, rows_acc=rows_acc,
                                       dbuf=(n_s + n_d) <= 12000)
            prev = out_den[0, :16]
            nr = out_den.shape[1]
            den4 = out_den.reshape(2, nr * 128)[:, :2 * rows_acc].reshape(
                2, rows_acc, 2).transpose(1, 0, 2).reshape(rows_acc, 4)
            acc[d].append((out_ch, den4))
            bias[d].append(p['bias'].reshape(1, _HID))
        nrm = params['norms'][l]
        g = nrm['g'].reshape(1, _HID)
        b = nrm['b'].reshape(1, _HID)
        x = {nt: _combine(x[nt], g, b, acc[nt], bias[nt], _BLK[nt])
             for nt in _NODE_ORDER}

    pooled = [_pool(x[nt], _BLK[nt]) for nt in _NODE_ORDER]
    return jnp.concatenate(pooled, axis=-1)
